# Initial kernel scaffold; baseline (speedup 1.0000x reference)
import functools

import jax
import jax.numpy as jnp
from jax import lax
from jax.experimental import pallas as pl
from jax.experimental.pallas import tpu as pltpu
from jax.experimental.pallas import tpu_sc as plsc

N = 10000
E = 320000
IN_DIM = 128
HID = 32
HEADS = 8
OUT_DIM = 64

ROW1 = 144
ROW2 = 80
CH = 80
NEG = 0.2
RPT = N // 16

_mesh = plsc.VectorSubcoreMesh(core_axis_name="c", subcore_axis_name="s")


_B = 2000


def _prep1_body(x_ref, w_ref, as_ref, ad_ref, tlo_ref, thi_ref, td_ref):
    h = jnp.dot(x_ref[...], w_ref[...], preferred_element_type=jnp.float32)
    asrc = jnp.dot(h, as_ref[...], preferred_element_type=jnp.float32)
    adst = jnp.dot(h, ad_ref[...], preferred_element_type=jnp.float32)
    z12 = jnp.zeros((_B, 12), jnp.float32)
    tlo_ref[...] = jnp.concatenate([h[:, :128], asrc[:, 0:4], z12], axis=1)
    thi_ref[...] = jnp.concatenate([h[:, 128:], asrc[:, 4:8], z12], axis=1)
    td_ref[...] = jnp.concatenate([adst, jnp.zeros((_B, 8), jnp.float32)], axis=1)


def _prep1(x, w1, as1, ad1):
    return pl.pallas_call(
        _prep1_body,
        grid=(N // _B,),
        in_specs=[
            pl.BlockSpec((_B, IN_DIM), lambda i: (i, 0)),
            pl.BlockSpec((IN_DIM, 256), lambda i: (0, 0)),
            pl.BlockSpec((256, 8), lambda i: (0, 0)),
            pl.BlockSpec((256, 8), lambda i: (0, 0)),
        ],
        out_specs=[
            pl.BlockSpec((_B, ROW1), lambda i: (i, 0)),
            pl.BlockSpec((_B, ROW1), lambda i: (i, 0)),
            pl.BlockSpec((_B, 16), lambda i: (i, 0)),
        ],
        out_shape=[
            jax.ShapeDtypeStruct((N, ROW1), jnp.float32),
            jax.ShapeDtypeStruct((N, ROW1), jnp.float32),
            jax.ShapeDtypeStruct((N, 16), jnp.float32),
        ],
    )(x, w1, as1, ad1)


def _mid_body(alo_ref, ahi_ref, tlo_ref, thi_ref, td_ref, w2_ref, e1_ref,
              as2_ref, ad2_ref, b1_ref, t2_ref, td2_ref):
    asrc8 = jnp.concatenate([tlo_ref[:, 128:132], thi_ref[:, 128:132]], axis=1)
    adst8 = td_ref[:, 0:8]
    a = asrc8 + adst8
    wself = jnp.exp(jnp.where(a > 0, a, NEG * a))
    h1 = jnp.concatenate([tlo_ref[:, 0:128], thi_ref[:, 0:128]], axis=1)
    num = jnp.concatenate([alo_ref[:, 0:128], ahi_ref[:, 0:128]], axis=1)
    den8 = jnp.concatenate([alo_ref[:, 128:132], ahi_ref[:, 128:132]], axis=1)
    den8 = den8 + wself + 1e-16
    e1 = e1_ref[...]
    num = num + jnp.dot(wself, e1, preferred_element_type=jnp.float32) * h1
    den = jnp.dot(den8, e1, preferred_element_type=jnp.float32)
    out1 = jnp.maximum(num / den + b1_ref[0:1, :], 0.0)
    h2 = jnp.dot(out1, w2_ref[...], preferred_element_type=jnp.float32)
    asrc2 = jnp.sum(h2 * as2_ref[0:1, :], axis=1, keepdims=True)
    adst2 = jnp.sum(h2 * ad2_ref[0:1, :], axis=1, keepdims=True)
    z15 = jnp.zeros((_B, 15), jnp.float32)
    t2_ref[...] = jnp.concatenate([h2, asrc2, z15], axis=1)
    td2_ref[...] = jnp.concatenate([adst2, z15], axis=1)


def _mid(alo, ahi, tlo, thi, td1, w2, e1, as2, ad2, b1):
    return pl.pallas_call(
        _mid_body,
        grid=(N // _B,),
        in_specs=[
            pl.BlockSpec((_B, ROW1), lambda i: (i, 0)),
            pl.BlockSpec((_B, ROW1), lambda i: (i, 0)),
            pl.BlockSpec((_B, ROW1), lambda i: (i, 0)),
            pl.BlockSpec((_B, ROW1), lambda i: (i, 0)),
            pl.BlockSpec((_B, 16), lambda i: (i, 0)),
            pl.BlockSpec((256, OUT_DIM), lambda i: (0, 0)),
            pl.BlockSpec((8, 256), lambda i: (0, 0)),
            pl.BlockSpec((8, OUT_DIM), lambda i: (0, 0)),
            pl.BlockSpec((8, OUT_DIM), lambda i: (0, 0)),
            pl.BlockSpec((8, 256), lambda i: (0, 0)),
        ],
        out_specs=[
            pl.BlockSpec((_B, ROW2), lambda i: (i, 0)),
            pl.BlockSpec((_B, 16), lambda i: (i, 0)),
        ],
        out_shape=[
            jax.ShapeDtypeStruct((N, ROW2), jnp.float32),
            jax.ShapeDtypeStruct((N, 16), jnp.float32),
        ],
    )(alo, ahi, tlo, thi, td1, w2, e1, as2, ad2, b1)


def _final_body(a2lo_ref, a2hi_ref, t2_ref, td2_ref, b2_ref, out_ref):
    num = a2lo_ref[:, 0:64] + a2hi_ref[:, 0:64]
    den = a2lo_ref[:, 64:65] + a2hi_ref[:, 64:65]
    h2 = t2_ref[:, 0:64]
    a = t2_ref[:, 64:65] + td2_ref[:, 0:1]
    w = jnp.exp(jnp.where(a > 0, a, NEG * a))
    out_ref[...] = (num + w * h2) / (den + w + 1e-16) + b2_ref[0:1, :]


def _final(a2lo, a2hi, t2, td2, b2):
    return pl.pallas_call(
        _final_body,
        grid=(N // _B,),
        in_specs=[
            pl.BlockSpec((_B, ROW2), lambda i: (i, 0)),
            pl.BlockSpec((_B, ROW2), lambda i: (i, 0)),
            pl.BlockSpec((_B, ROW2), lambda i: (i, 0)),
            pl.BlockSpec((_B, 16), lambda i: (i, 0)),
            pl.BlockSpec((8, OUT_DIM), lambda i: (0, 0)),
        ],
        out_specs=pl.BlockSpec((_B, OUT_DIM), lambda i: (i, 0)),
        out_shape=jax.ShapeDtypeStruct((N, OUT_DIM), jnp.float32),
    )(a2lo, a2hi, t2, td2, b2)


def _zero_rows(buf, rows, width):
    z = jnp.zeros((16,), jnp.float32)

    def body(r, _):
        for v in range(width // 16):
            buf[r, pl.ds(16 * v, 16)] = z
        return 0

    lax.fori_loop(0, rows, body, 0)


def _zero_acc(gbuf, acc, base_r):
    for k in range(RPT // CH):
        pltpu.sync_copy(gbuf.at[pl.ds(0, CH)], acc.at[pl.ds(base_r + CH * k, CH)])
    rem = RPT - (RPT // CH) * CH
    if rem:
        pltpu.sync_copy(gbuf.at[pl.ds(0, rem)],
                        acc.at[pl.ds(base_r + RPT - rem, rem)])


@functools.partial(
    pl.kernel,
    out_type=jax.ShapeDtypeStruct((2 * N, ROW1), jnp.float32),
    mesh=_mesh,
    scratch_types=[
        pltpu.VMEM((CH, ROW1), jnp.float32),
        pltpu.VMEM((CH, 16), jnp.float32),
        pltpu.VMEM((CH,), jnp.int32),
        pltpu.VMEM((CH,), jnp.int32),
        pltpu.VMEM_SHARED((N, ROW1), jnp.float32),
        pltpu.SemaphoreType.DMA,
    ],
)
def _sc1(t1, td, srcv, dstv, out, gbuf, dbuf, sibuf, dibuf, acc, sem):
    c = lax.axis_index("c")
    s = lax.axis_index("s")
    _zero_rows(gbuf, CH, ROW1)
    _zero_acc(gbuf, acc, s * RPT)
    plsc.subcore_barrier()

    lane = lax.iota(jnp.int32, 16)
    shift_idx = (lane + 4 * c) & 15
    head_idx = [jnp.full((16,), hh, jnp.int32) for hh in range(4)]
    off = c * N

    ept = E // 16
    ebase = s * ept

    def chunk(j, _):
        be = pl.multiple_of(ebase + j * CH, 8)
        pltpu.sync_copy(srcv.at[pl.ds(be, CH)], sibuf)
        pltpu.sync_copy(dstv.at[pl.ds(be, CH)], dibuf)
        for k in range(CH // 16):
            sibuf[pl.ds(16 * k, 16)] = sibuf[pl.ds(16 * k, 16)] + off
        pltpu.async_copy(t1.at[sibuf], gbuf, sem).wait()
        pltpu.async_copy(td.at[dibuf], dbuf, sem).wait()

        def edge(e, _):
            adsh = jnp.take(dbuf[e, pl.ds(0, 16)], shift_idx,
                            mode="promise_in_bounds")
            a = gbuf[e, pl.ds(128, 16)] + adsh
            a = jnp.where(a > 0, a, NEG * a)
            w = jnp.exp(a)
            gbuf[e, pl.ds(128, 16)] = jnp.where(lane < 4, w, 0.0)
            for hh in range(4):
                ws = jnp.take(w, head_idx[hh], mode="promise_in_bounds")
                for v in (2 * hh, 2 * hh + 1):
                    gbuf[e, pl.ds(16 * v, 16)] = gbuf[e, pl.ds(16 * v, 16)] * ws
            return 0

        lax.fori_loop(0, CH, edge, 0)
        pltpu.sync_copy(gbuf, acc.at[dibuf], add=True)
        return 0

    lax.fori_loop(0, ept // CH, chunk, 0)
    plsc.subcore_barrier()
    pltpu.sync_copy(acc.at[pl.ds(s * RPT, RPT)],
                    out.at[pl.ds(c * N + s * RPT, RPT)])


@functools.partial(
    pl.kernel,
    out_type=jax.ShapeDtypeStruct((2 * N, ROW2), jnp.float32),
    mesh=_mesh,
    scratch_types=[
        pltpu.VMEM((CH, ROW2), jnp.float32),
        pltpu.VMEM((CH, 16), jnp.float32),
        pltpu.VMEM((CH,), jnp.int32),
        pltpu.VMEM((CH,), jnp.int32),
        pltpu.VMEM_SHARED((N, ROW2), jnp.float32),
        pltpu.SemaphoreType.DMA,
    ],
)
def _sc2(t2, td2, srcv, dstv, out, gbuf, dbuf, sibuf, dibuf, acc, sem):
    c = lax.axis_index("c")
    s = lax.axis_index("s")
    _zero_rows(gbuf, CH, ROW2)
    _zero_acc(gbuf, acc, s * RPT)
    plsc.subcore_barrier()

    lane = lax.iota(jnp.int32, 16)
    zidx = jnp.zeros((16,), jnp.int32)

    ept = E // 32
    ebase = (s * 2 + c) * ept

    def chunk(j, _):
        be = pl.multiple_of(ebase + j * CH, 8)
        pltpu.sync_copy(srcv.at[pl.ds(be, CH)], sibuf)
        pltpu.sync_copy(dstv.at[pl.ds(be, CH)], dibuf)
        pltpu.async_copy(t2.at[sibuf], gbuf, sem).wait()
        pltpu.async_copy(td2.at[dibuf], dbuf, sem).wait()

        def edge(e, _):
            a = gbuf[e, pl.ds(64, 16)] + dbuf[e, pl.ds(0, 16)]
            a = jnp.where(a > 0, a, NEG * a)
            w = jnp.exp(a)
            gbuf[e, pl.ds(64, 16)] = jnp.where(lane < 1, w, 0.0)
            ws = jnp.take(w, zidx, mode="promise_in_bounds")
            for v in range(4):
                gbuf[e, pl.ds(16 * v, 16)] = gbuf[e, pl.ds(16 * v, 16)] * ws
            return 0

        lax.fori_loop(0, CH, edge, 0)
        pltpu.sync_copy(gbuf, acc.at[dibuf], add=True)
        return 0

    lax.fori_loop(0, ept // CH, chunk, 0)
    plsc.subcore_barrier()
    pltpu.sync_copy(acc.at[pl.ds(s * RPT, RPT)],
                    out.at[pl.ds(c * N + s * RPT, RPT)])


def kernel(x, edge_index, W1, att_src1, att_dst1, bias1,
           W2, att_src2, att_dst2, bias2):
    src = edge_index[0]
    dst = edge_index[1]

    e1 = (jnp.arange(256, dtype=jnp.int32) // HID ==
          jnp.arange(HEADS, dtype=jnp.int32)[:, None]).astype(jnp.float32)
    as1 = e1.T * att_src1.reshape(256)[:, None]
    ad1 = e1.T * att_dst1.reshape(256)[:, None]
    as2 = jnp.broadcast_to(att_src2.reshape(1, OUT_DIM), (8, OUT_DIM))
    ad2 = jnp.broadcast_to(att_dst2.reshape(1, OUT_DIM), (8, OUT_DIM))
    b1 = jnp.broadcast_to(bias1.reshape(1, 256), (8, 256))
    b2 = jnp.broadcast_to(bias2.reshape(1, OUT_DIM), (8, OUT_DIM))

    tlo, thi, td1 = _prep1(x, W1, as1, ad1)
    t1 = jnp.concatenate([tlo, thi], axis=0)
    acc1 = _sc1(t1, td1, src, dst)
    t2, td2 = _mid(acc1[:N], acc1[N:], tlo, thi, td1, W2, e1, as2, ad2, b1)
    acc2 = _sc2(t2, td2, src, dst)
    return _final(acc2[:N], acc2[N:], t2, td2, b2)


# speedup vs baseline: 28.3738x; 28.3738x over previous
import functools

import jax
import jax.numpy as jnp
from jax import lax
from jax.experimental import pallas as pl
from jax.experimental.pallas import tpu as pltpu
from jax.experimental.pallas import tpu_sc as plsc

N = 10000
E = 320000
IN_DIM = 128
HID = 32
HEADS = 8
OUT_DIM = 64

ROW1 = 144
ROW2 = 80
CH = 80
NEG = 0.2
RPT = N // 16

_mesh = plsc.VectorSubcoreMesh(core_axis_name="c", subcore_axis_name="s")

def _take16(vec, idx):
    return jnp.take_along_axis(vec, idx, axis=0)


_B = 2000


def _prep1_body(x_ref, w_ref, as_ref, ad_ref, tlo_ref, thi_ref, td_ref):
    h = jnp.dot(x_ref[...], w_ref[...], preferred_element_type=jnp.float32)
    asrc = jnp.dot(h, as_ref[...], preferred_element_type=jnp.float32)
    adst = jnp.dot(h, ad_ref[...], preferred_element_type=jnp.float32)
    z12 = jnp.zeros((_B, 12), jnp.float32)
    tlo_ref[...] = jnp.concatenate([h[:, :128], asrc[:, 0:4], z12], axis=1)
    thi_ref[...] = jnp.concatenate([h[:, 128:], asrc[:, 4:8], z12], axis=1)
    td_ref[...] = jnp.concatenate([adst, jnp.zeros((_B, 8), jnp.float32)], axis=1)


def _prep1(x, w1, as1, ad1):
    return pl.pallas_call(
        _prep1_body,
        grid=(N // _B,),
        in_specs=[
            pl.BlockSpec((_B, IN_DIM), lambda i: (i, 0)),
            pl.BlockSpec((IN_DIM, 256), lambda i: (0, 0)),
            pl.BlockSpec((256, 8), lambda i: (0, 0)),
            pl.BlockSpec((256, 8), lambda i: (0, 0)),
        ],
        out_specs=[
            pl.BlockSpec((_B, ROW1), lambda i: (i, 0)),
            pl.BlockSpec((_B, ROW1), lambda i: (i, 0)),
            pl.BlockSpec((_B, 16), lambda i: (i, 0)),
        ],
        out_shape=[
            jax.ShapeDtypeStruct((N, ROW1), jnp.float32),
            jax.ShapeDtypeStruct((N, ROW1), jnp.float32),
            jax.ShapeDtypeStruct((N, 16), jnp.float32),
        ],
    )(x, w1, as1, ad1)


def _mid_body(alo_ref, ahi_ref, tlo_ref, thi_ref, td_ref, w2_ref, e1_ref,
              as2_ref, ad2_ref, b1_ref, t2_ref, td2_ref):
    asrc8 = jnp.concatenate([tlo_ref[:, 128:132], thi_ref[:, 128:132]], axis=1)
    adst8 = td_ref[:, 0:8]
    a = asrc8 + adst8
    wself = jnp.exp(jnp.where(a > 0, a, NEG * a))
    h1 = jnp.concatenate([tlo_ref[:, 0:128], thi_ref[:, 0:128]], axis=1)
    num = jnp.concatenate([alo_ref[:, 0:128], ahi_ref[:, 0:128]], axis=1)
    den8 = jnp.concatenate([alo_ref[:, 128:132], ahi_ref[:, 128:132]], axis=1)
    den8 = den8 + wself + 1e-16
    e1 = e1_ref[...]
    num = num + jnp.dot(wself, e1, preferred_element_type=jnp.float32) * h1
    den = jnp.dot(den8, e1, preferred_element_type=jnp.float32)
    out1 = jnp.maximum(num / den + b1_ref[0:1, :], 0.0)
    h2 = jnp.dot(out1, w2_ref[...], preferred_element_type=jnp.float32)
    asrc2 = jnp.sum(h2 * as2_ref[0:1, :], axis=1, keepdims=True)
    adst2 = jnp.sum(h2 * ad2_ref[0:1, :], axis=1, keepdims=True)
    z15 = jnp.zeros((_B, 15), jnp.float32)
    t2_ref[...] = jnp.concatenate([h2, asrc2, z15], axis=1)
    td2_ref[...] = jnp.concatenate([adst2, z15], axis=1)


def _mid(alo, ahi, tlo, thi, td1, w2, e1, as2, ad2, b1):
    return pl.pallas_call(
        _mid_body,
        grid=(N // _B,),
        in_specs=[
            pl.BlockSpec((_B, ROW1), lambda i: (i, 0)),
            pl.BlockSpec((_B, ROW1), lambda i: (i, 0)),
            pl.BlockSpec((_B, ROW1), lambda i: (i, 0)),
            pl.BlockSpec((_B, ROW1), lambda i: (i, 0)),
            pl.BlockSpec((_B, 16), lambda i: (i, 0)),
            pl.BlockSpec((256, OUT_DIM), lambda i: (0, 0)),
            pl.BlockSpec((8, 256), lambda i: (0, 0)),
            pl.BlockSpec((8, OUT_DIM), lambda i: (0, 0)),
            pl.BlockSpec((8, OUT_DIM), lambda i: (0, 0)),
            pl.BlockSpec((8, 256), lambda i: (0, 0)),
        ],
        out_specs=[
            pl.BlockSpec((_B, ROW2), lambda i: (i, 0)),
            pl.BlockSpec((_B, 16), lambda i: (i, 0)),
        ],
        out_shape=[
            jax.ShapeDtypeStruct((N, ROW2), jnp.float32),
            jax.ShapeDtypeStruct((N, 16), jnp.float32),
        ],
    )(alo, ahi, tlo, thi, td1, w2, e1, as2, ad2, b1)


def _final_body(a2lo_ref, a2hi_ref, t2_ref, td2_ref, b2_ref, out_ref):
    num = a2lo_ref[:, 0:64] + a2hi_ref[:, 0:64]
    den = a2lo_ref[:, 64:65] + a2hi_ref[:, 64:65]
    h2 = t2_ref[:, 0:64]
    a = t2_ref[:, 64:65] + td2_ref[:, 0:1]
    w = jnp.exp(jnp.where(a > 0, a, NEG * a))
    out_ref[...] = (num + w * h2) / (den + w + 1e-16) + b2_ref[0:1, :]


def _final(a2lo, a2hi, t2, td2, b2):
    return pl.pallas_call(
        _final_body,
        grid=(N // _B,),
        in_specs=[
            pl.BlockSpec((_B, ROW2), lambda i: (i, 0)),
            pl.BlockSpec((_B, ROW2), lambda i: (i, 0)),
            pl.BlockSpec((_B, ROW2), lambda i: (i, 0)),
            pl.BlockSpec((_B, 16), lambda i: (i, 0)),
            pl.BlockSpec((8, OUT_DIM), lambda i: (0, 0)),
        ],
        out_specs=pl.BlockSpec((_B, OUT_DIM), lambda i: (i, 0)),
        out_shape=jax.ShapeDtypeStruct((N, OUT_DIM), jnp.float32),
    )(a2lo, a2hi, t2, td2, b2)


def _zero_rows(buf, rows, width):
    z = jnp.zeros((16,), jnp.float32)

    def body(r, _):
        for v in range(width // 16):
            buf[r, pl.ds(16 * v, 16)] = z
        return 0

    lax.fori_loop(0, rows, body, 0)


def _zero_acc(gbuf, acc, base_r):
    for k in range(RPT // CH):
        pltpu.sync_copy(gbuf.at[pl.ds(0, CH)], acc.at[pl.ds(base_r + CH * k, CH)])
    rem = RPT - (RPT // CH) * CH
    if rem:
        pltpu.sync_copy(gbuf.at[pl.ds(0, rem)],
                        acc.at[pl.ds(base_r + RPT - rem, rem)])


@functools.partial(
    pl.kernel,
    out_type=jax.ShapeDtypeStruct((2 * N, ROW1), jnp.float32),
    mesh=_mesh,
    compiler_params=pltpu.CompilerParams(use_tc_tiling_on_sc=False),
    scratch_types=[
        pltpu.VMEM((CH, ROW1), jnp.float32),
        pltpu.VMEM((CH, 16), jnp.float32),
        pltpu.VMEM((CH,), jnp.int32),
        pltpu.VMEM((CH,), jnp.int32),
        pltpu.VMEM_SHARED((N, ROW1), jnp.float32),
        pltpu.SemaphoreType.DMA,
    ],
)
def _sc1(t1, td, srcv, dstv, out, gbuf, dbuf, sibuf, dibuf, acc, sem):
    c = lax.axis_index("c")
    s = lax.axis_index("s")
    _zero_rows(gbuf, CH, ROW1)
    _zero_acc(gbuf, acc, s * RPT)
    plsc.subcore_barrier()

    lane = lax.iota(jnp.int32, 16)
    shift_idx = (lane + 4 * c) & 15
    head_idx = [jnp.full((16,), hh, jnp.int32) for hh in range(4)]
    off = c * N

    ept = E // 16
    ebase = s * ept

    def chunk(j, _):
        be = pl.multiple_of(ebase + j * CH, 8)
        pltpu.sync_copy(srcv.at[pl.ds(be, CH)], sibuf)
        pltpu.sync_copy(dstv.at[pl.ds(be, CH)], dibuf)
        for k in range(CH // 16):
            sibuf[pl.ds(16 * k, 16)] = sibuf[pl.ds(16 * k, 16)] + off
        pltpu.async_copy(t1.at[sibuf], gbuf, sem).wait()
        pltpu.async_copy(td.at[dibuf], dbuf, sem).wait()

        def edge(e, _):
            adsh = _take16(dbuf[e, pl.ds(0, 16)], shift_idx)
            a = gbuf[e, pl.ds(128, 16)] + adsh
            a = jnp.where(a > 0, a, NEG * a)
            w = jnp.exp(a)
            gbuf[e, pl.ds(128, 16)] = jnp.where(lane < 4, w, 0.0)
            for hh in range(4):
                ws = _take16(w, head_idx[hh])
                for v in (2 * hh, 2 * hh + 1):
                    gbuf[e, pl.ds(16 * v, 16)] = gbuf[e, pl.ds(16 * v, 16)] * ws
            return 0

        lax.fori_loop(0, CH, edge, 0)
        pltpu.sync_copy(gbuf, acc.at[dibuf], add=True)
        return 0

    lax.fori_loop(0, ept // CH, chunk, 0)
    plsc.subcore_barrier()
    pltpu.sync_copy(acc.at[pl.ds(s * RPT, RPT)],
                    out.at[pl.ds(c * N + s * RPT, RPT)])


@functools.partial(
    pl.kernel,
    out_type=jax.ShapeDtypeStruct((2 * N, ROW2), jnp.float32),
    mesh=_mesh,
    compiler_params=pltpu.CompilerParams(use_tc_tiling_on_sc=False),
    scratch_types=[
        pltpu.VMEM((CH, ROW2), jnp.float32),
        pltpu.VMEM((CH, 16), jnp.float32),
        pltpu.VMEM((CH,), jnp.int32),
        pltpu.VMEM((CH,), jnp.int32),
        pltpu.VMEM_SHARED((N, ROW2), jnp.float32),
        pltpu.SemaphoreType.DMA,
    ],
)
def _sc2(t2, td2, srcv, dstv, out, gbuf, dbuf, sibuf, dibuf, acc, sem):
    c = lax.axis_index("c")
    s = lax.axis_index("s")
    _zero_rows(gbuf, CH, ROW2)
    _zero_acc(gbuf, acc, s * RPT)
    plsc.subcore_barrier()

    lane = lax.iota(jnp.int32, 16)
    zidx = jnp.zeros((16,), jnp.int32)

    ept = E // 32
    ebase = (s * 2 + c) * ept

    def chunk(j, _):
        be = pl.multiple_of(ebase + j * CH, 8)
        pltpu.sync_copy(srcv.at[pl.ds(be, CH)], sibuf)
        pltpu.sync_copy(dstv.at[pl.ds(be, CH)], dibuf)
        pltpu.async_copy(t2.at[sibuf], gbuf, sem).wait()
        pltpu.async_copy(td2.at[dibuf], dbuf, sem).wait()

        def edge(e, _):
            a = gbuf[e, pl.ds(64, 16)] + dbuf[e, pl.ds(0, 16)]
            a = jnp.where(a > 0, a, NEG * a)
            w = jnp.exp(a)
            gbuf[e, pl.ds(64, 16)] = jnp.where(lane < 1, w, 0.0)
            ws = _take16(w, zidx)
            for v in range(4):
                gbuf[e, pl.ds(16 * v, 16)] = gbuf[e, pl.ds(16 * v, 16)] * ws
            return 0

        lax.fori_loop(0, CH, edge, 0)
        pltpu.sync_copy(gbuf, acc.at[dibuf], add=True)
        return 0

    lax.fori_loop(0, ept // CH, chunk, 0)
    plsc.subcore_barrier()
    pltpu.sync_copy(acc.at[pl.ds(s * RPT, RPT)],
                    out.at[pl.ds(c * N + s * RPT, RPT)])


def kernel(x, edge_index, W1, att_src1, att_dst1, bias1,
           W2, att_src2, att_dst2, bias2):
    src = edge_index[0]
    dst = edge_index[1]

    e1 = (jnp.arange(256, dtype=jnp.int32) // HID ==
          jnp.arange(HEADS, dtype=jnp.int32)[:, None]).astype(jnp.float32)
    as1 = e1.T * att_src1.reshape(256)[:, None]
    ad1 = e1.T * att_dst1.reshape(256)[:, None]
    as2 = jnp.broadcast_to(att_src2.reshape(1, OUT_DIM), (8, OUT_DIM))
    ad2 = jnp.broadcast_to(att_dst2.reshape(1, OUT_DIM), (8, OUT_DIM))
    b1 = jnp.broadcast_to(bias1.reshape(1, 256), (8, 256))
    b2 = jnp.broadcast_to(bias2.reshape(1, OUT_DIM), (8, OUT_DIM))

    tlo, thi, td1 = _prep1(x, W1, as1, ad1)
    t1 = jnp.concatenate([tlo, thi], axis=0)
    acc1 = _sc1(t1, td1, src, dst)
    t2, td2 = _mid(acc1[:N], acc1[N:], tlo, thi, td1, W2, e1, as2, ad2, b1)
    acc2 = _sc2(t2, td2, src, dst)
    return _final(acc2[:N], acc2[N:], t2, td2, b2)


# speedup vs baseline: 56.4303x; 1.9888x over previous
import functools

import jax
import jax.numpy as jnp
from jax import lax
from jax.experimental import pallas as pl
from jax.experimental.pallas import tpu as pltpu
from jax.experimental.pallas import tpu_sc as plsc

N = 10000
E = 320000
IN_DIM = 128
HID = 32
HEADS = 8
OUT_DIM = 64

ROW1 = 144
ROW2 = 80
CH = 80
NEG = 0.2
RPT = N // 16

_mesh = plsc.VectorSubcoreMesh(core_axis_name="c", subcore_axis_name="s")

def _take16(vec, idx):
    return jnp.take_along_axis(vec, idx, axis=0)


_B = 2000


def _prep1_body(x_ref, w_ref, as_ref, ad_ref, tlo_ref, thi_ref, td_ref):
    h = jnp.dot(x_ref[...], w_ref[...], preferred_element_type=jnp.float32)
    asrc = jnp.dot(h, as_ref[...], preferred_element_type=jnp.float32)
    adst = jnp.dot(h, ad_ref[...], preferred_element_type=jnp.float32)
    z12 = jnp.zeros((_B, 12), jnp.float32)
    tlo_ref[...] = jnp.concatenate([h[:, :128], asrc[:, 0:4], z12], axis=1)
    thi_ref[...] = jnp.concatenate([h[:, 128:], asrc[:, 4:8], z12], axis=1)
    td_ref[...] = jnp.concatenate([adst, jnp.zeros((_B, 8), jnp.float32)], axis=1)


def _prep1(x, w1, as1, ad1):
    return pl.pallas_call(
        _prep1_body,
        grid=(N // _B,),
        in_specs=[
            pl.BlockSpec((_B, IN_DIM), lambda i: (i, 0)),
            pl.BlockSpec((IN_DIM, 256), lambda i: (0, 0)),
            pl.BlockSpec((256, 8), lambda i: (0, 0)),
            pl.BlockSpec((256, 8), lambda i: (0, 0)),
        ],
        out_specs=[
            pl.BlockSpec((_B, ROW1), lambda i: (i, 0)),
            pl.BlockSpec((_B, ROW1), lambda i: (i, 0)),
            pl.BlockSpec((_B, 16), lambda i: (i, 0)),
        ],
        out_shape=[
            jax.ShapeDtypeStruct((N, ROW1), jnp.float32),
            jax.ShapeDtypeStruct((N, ROW1), jnp.float32),
            jax.ShapeDtypeStruct((N, 16), jnp.float32),
        ],
    )(x, w1, as1, ad1)


def _mid_body(alo_ref, ahi_ref, tlo_ref, thi_ref, td_ref, w2_ref, e1_ref,
              as2_ref, ad2_ref, b1_ref, t2_ref, td2_ref):
    asrc8 = jnp.concatenate([tlo_ref[:, 128:132], thi_ref[:, 128:132]], axis=1)
    adst8 = td_ref[:, 0:8]
    a = asrc8 + adst8
    wself = jnp.exp(jnp.where(a > 0, a, NEG * a))
    h1 = jnp.concatenate([tlo_ref[:, 0:128], thi_ref[:, 0:128]], axis=1)
    num = jnp.concatenate([alo_ref[:, 0:128], ahi_ref[:, 0:128]], axis=1)
    den8 = jnp.concatenate([alo_ref[:, 128:132], ahi_ref[:, 128:132]], axis=1)
    den8 = den8 + wself + 1e-16
    e1 = e1_ref[...]
    num = num + jnp.dot(wself, e1, preferred_element_type=jnp.float32) * h1
    den = jnp.dot(den8, e1, preferred_element_type=jnp.float32)
    out1 = jnp.maximum(num / den + b1_ref[0:1, :], 0.0)
    h2 = jnp.dot(out1, w2_ref[...], preferred_element_type=jnp.float32)
    asrc2 = jnp.sum(h2 * as2_ref[0:1, :], axis=1, keepdims=True)
    adst2 = jnp.sum(h2 * ad2_ref[0:1, :], axis=1, keepdims=True)
    z15 = jnp.zeros((_B, 15), jnp.float32)
    t2_ref[...] = jnp.concatenate([h2, asrc2, z15], axis=1)
    td2_ref[...] = jnp.concatenate([adst2, z15], axis=1)


def _mid(alo, ahi, tlo, thi, td1, w2, e1, as2, ad2, b1):
    return pl.pallas_call(
        _mid_body,
        grid=(N // _B,),
        in_specs=[
            pl.BlockSpec((_B, ROW1), lambda i: (i, 0)),
            pl.BlockSpec((_B, ROW1), lambda i: (i, 0)),
            pl.BlockSpec((_B, ROW1), lambda i: (i, 0)),
            pl.BlockSpec((_B, ROW1), lambda i: (i, 0)),
            pl.BlockSpec((_B, 16), lambda i: (i, 0)),
            pl.BlockSpec((256, OUT_DIM), lambda i: (0, 0)),
            pl.BlockSpec((8, 256), lambda i: (0, 0)),
            pl.BlockSpec((8, OUT_DIM), lambda i: (0, 0)),
            pl.BlockSpec((8, OUT_DIM), lambda i: (0, 0)),
            pl.BlockSpec((8, 256), lambda i: (0, 0)),
        ],
        out_specs=[
            pl.BlockSpec((_B, ROW2), lambda i: (i, 0)),
            pl.BlockSpec((_B, 16), lambda i: (i, 0)),
        ],
        out_shape=[
            jax.ShapeDtypeStruct((N, ROW2), jnp.float32),
            jax.ShapeDtypeStruct((N, 16), jnp.float32),
        ],
    )(alo, ahi, tlo, thi, td1, w2, e1, as2, ad2, b1)


def _final_body(a2lo_ref, a2hi_ref, t2_ref, td2_ref, b2_ref, out_ref):
    num = a2lo_ref[:, 0:64] + a2hi_ref[:, 0:64]
    den = a2lo_ref[:, 64:65] + a2hi_ref[:, 64:65]
    h2 = t2_ref[:, 0:64]
    a = t2_ref[:, 64:65] + td2_ref[:, 0:1]
    w = jnp.exp(jnp.where(a > 0, a, NEG * a))
    out_ref[...] = (num + w * h2) / (den + w + 1e-16) + b2_ref[0:1, :]


def _final(a2lo, a2hi, t2, td2, b2):
    return pl.pallas_call(
        _final_body,
        grid=(N // _B,),
        in_specs=[
            pl.BlockSpec((_B, ROW2), lambda i: (i, 0)),
            pl.BlockSpec((_B, ROW2), lambda i: (i, 0)),
            pl.BlockSpec((_B, ROW2), lambda i: (i, 0)),
            pl.BlockSpec((_B, 16), lambda i: (i, 0)),
            pl.BlockSpec((8, OUT_DIM), lambda i: (0, 0)),
        ],
        out_specs=pl.BlockSpec((_B, OUT_DIM), lambda i: (i, 0)),
        out_shape=jax.ShapeDtypeStruct((N, OUT_DIM), jnp.float32),
    )(a2lo, a2hi, t2, td2, b2)


def _zero_rows(buf, rows, width):
    z = jnp.zeros((16,), jnp.float32)

    def body(r, _):
        for v in range(width // 16):
            buf[r, pl.ds(16 * v, 16)] = z
        return 0

    lax.fori_loop(0, rows, body, 0)


def _zero_acc(gbuf, acc, base_r):
    for k in range(RPT // CH):
        pltpu.sync_copy(gbuf.at[pl.ds(0, CH)], acc.at[pl.ds(base_r + CH * k, CH)])
    rem = RPT - (RPT // CH) * CH
    if rem:
        pltpu.sync_copy(gbuf.at[pl.ds(0, rem)],
                        acc.at[pl.ds(base_r + RPT - rem, rem)])


def _make_sc(row, split_features, edge_fn):
    ept = E // 16 if split_features else E // 32
    nch = ept // CH

    @functools.partial(
        pl.kernel,
        out_type=jax.ShapeDtypeStruct((2 * N, row), jnp.float32),
        mesh=_mesh,
        compiler_params=pltpu.CompilerParams(use_tc_tiling_on_sc=False),
        scratch_types=[
            pltpu.VMEM((CH,), jnp.int32),
            pltpu.VMEM((CH,), jnp.int32),
            pltpu.VMEM((CH,), jnp.int32),
            pltpu.VMEM((CH,), jnp.int32),
            pltpu.VMEM((CH,), jnp.int32),
            pltpu.VMEM((CH,), jnp.int32),
            pltpu.VMEM((CH, row), jnp.float32),
            pltpu.VMEM((CH, row), jnp.float32),
            pltpu.VMEM((CH, 16), jnp.float32),
            pltpu.VMEM((CH, 16), jnp.float32),
            pltpu.VMEM_SHARED((N, row), jnp.float32),
            pltpu.SemaphoreType.DMA,
            pltpu.SemaphoreType.DMA,
            pltpu.SemaphoreType.DMA,
            pltpu.SemaphoreType.DMA,
            pltpu.SemaphoreType.DMA,
            pltpu.SemaphoreType.DMA,
            pltpu.SemaphoreType.DMA,
            pltpu.SemaphoreType.DMA,
        ],
    )
    def sc_pass(tbl, td, srcv, dstv, out, si0, si1, di0, di1, dc0, dc1,
                gbuf0, gbuf1, dbuf0, dbuf1, acc,
                gs0, gs1, ds0, ds1, ss0, ss1, is0, is1):
        c = lax.axis_index("c")
        s = lax.axis_index("s")
        sib, dib, dsc = (si0, si1), (di0, di1), (dc0, dc1)
        gbufs, dbufs = (gbuf0, gbuf1), (dbuf0, dbuf1)
        gsems, dsems, ssems, isems = (gs0, gs1), (ds0, ds1), (ss0, ss1), (is0, is1)

        tile_e0 = (s * ept) if split_features else ((s * 2 + c) * ept)
        tile_e0 = pl.multiple_of(tile_e0, 8)

        _zero_rows(gbuf0, CH, row)
        _zero_acc(gbuf0, acc, s * RPT)
        plsc.subcore_barrier()

        off = c * N if split_features else 0

        def start_idx(j, b):
            e0 = pl.multiple_of(tile_e0 + j * CH, 8)
            pltpu.async_copy(srcv.at[pl.ds(e0, CH)], sib[b], isems[b])
            pltpu.async_copy(dstv.at[pl.ds(e0, CH)], dib[b], isems[b])

        def wait_idx_adjust(j, b):
            e0 = pl.multiple_of(tile_e0 + j * CH, 8)
            pltpu.make_async_copy(srcv.at[pl.ds(e0, CH)], sib[b],
                                  isems[b]).wait()
            pltpu.make_async_copy(dstv.at[pl.ds(e0, CH)], dib[b],
                                  isems[b]).wait()
            if split_features:
                for k in range(CH // 16):
                    sib[b][pl.ds(k * 16, 16)] = sib[b][pl.ds(k * 16, 16)] + off

        def start_gather(j, b):
            pltpu.async_copy(tbl.at[sib[b]], gbufs[b], gsems[b])
            pltpu.async_copy(td.at[dib[b]], dbufs[b], dsems[b])

        def wait_gather(j, b):
            pltpu.make_async_copy(tbl.at[sib[b]], gbufs[b], gsems[b]).wait()
            pltpu.make_async_copy(td.at[dib[b]], dbufs[b], dsems[b]).wait()

        def start_scatter(j, b):
            pltpu.async_copy(gbufs[b], acc.at[dsc[b]], ssems[b], add=True)

        def wait_scatter(j, b):
            pltpu.make_async_copy(gbufs[b], acc.at[dsc[b]], ssems[b]).wait()

        def keep_scatter_idx(b):
            for k in range(CH // 16):
                dsc[b][pl.ds(k * 16, 16)] = dib[b][pl.ds(k * 16, 16)]

        lane = lax.iota(jnp.int32, 16)
        consts = (lane, c)

        pltpu.sync_copy(srcv.at[pl.ds(tile_e0, CH)], si0)
        pltpu.sync_copy(dstv.at[pl.ds(tile_e0, CH)], di0)
        if split_features:
            for k in range(CH // 16):
                si0[pl.ds(k * 16, 16)] = si0[pl.ds(k * 16, 16)] + off
        start_gather(0, 0)
        start_idx(1, 1)

        def pair(jj, _):
            for b in (0, 1):
                j = 2 * jj + b

                @pl.when(j >= 1)
                def _():
                    wait_scatter(j - 1, 1 - b)

                @pl.when(j + 1 < nch)
                def _():
                    wait_idx_adjust(j + 1, 1 - b)
                    start_gather(j + 1, 1 - b)

                wait_gather(j, b)
                keep_scatter_idx(b)

                @pl.when(j + 2 < nch)
                def _():
                    start_idx(j + 2, b)

                edge_fn(gbufs[b], dbufs[b], consts)
                start_scatter(j, b)
            return 0

        lax.fori_loop(0, nch // 2, pair, 0)
        if nch % 2:
            jl = nch - 1
            wait_scatter(jl - 1, 1)
            wait_gather(jl, 0)
            keep_scatter_idx(0)
            edge_fn(gbufs[0], dbufs[0], consts)
            start_scatter(jl, 0)
            wait_scatter(jl, 0)
        else:
            wait_scatter(nch - 1, 1)
        plsc.subcore_barrier()
        pltpu.sync_copy(acc.at[pl.ds(s * RPT, RPT)],
                        out.at[pl.ds(c * N + s * RPT, RPT)])

    return sc_pass


def _edges1(gbuf, dbuf, consts):
    lane, c = consts
    shift_idx = (lane + 4 * c) & 15
    head_idx = [jnp.full((16,), hh, jnp.int32) for hh in range(4)]

    def edge(e, _):
        adsh = _take16(dbuf[e, pl.ds(0, 16)], shift_idx)
        a = gbuf[e, pl.ds(128, 16)] + adsh
        a = jnp.where(a > 0, a, NEG * a)
        w = jnp.exp(a)
        gbuf[e, pl.ds(128, 16)] = jnp.where(lane < 4, w, 0.0)
        for hh in range(4):
            ws = _take16(w, head_idx[hh])
            for v in (2 * hh, 2 * hh + 1):
                gbuf[e, pl.ds(16 * v, 16)] = gbuf[e, pl.ds(16 * v, 16)] * ws
        return 0

    lax.fori_loop(0, CH, edge, 0)


def _edges2(gbuf, dbuf, consts):
    lane, _ = consts
    zidx = jnp.zeros((16,), jnp.int32)

    def edge(e, _):
        a = gbuf[e, pl.ds(64, 16)] + dbuf[e, pl.ds(0, 16)]
        a = jnp.where(a > 0, a, NEG * a)
        w = jnp.exp(a)
        gbuf[e, pl.ds(64, 16)] = jnp.where(lane < 1, w, 0.0)
        ws = _take16(w, zidx)
        for v in range(4):
            gbuf[e, pl.ds(16 * v, 16)] = gbuf[e, pl.ds(16 * v, 16)] * ws
        return 0

    lax.fori_loop(0, CH, edge, 0)


_sc1 = _make_sc(ROW1, True, _edges1)
_sc2 = _make_sc(ROW2, False, _edges2)


def kernel(x, edge_index, W1, att_src1, att_dst1, bias1,
           W2, att_src2, att_dst2, bias2):
    src = edge_index[0]
    dst = edge_index[1]
    dst2d = dst.reshape(E // CH, CH)

    e1 = (jnp.arange(256, dtype=jnp.int32) // HID ==
          jnp.arange(HEADS, dtype=jnp.int32)[:, None]).astype(jnp.float32)
    as1 = e1.T * att_src1.reshape(256)[:, None]
    ad1 = e1.T * att_dst1.reshape(256)[:, None]
    as2 = jnp.broadcast_to(att_src2.reshape(1, OUT_DIM), (8, OUT_DIM))
    ad2 = jnp.broadcast_to(att_dst2.reshape(1, OUT_DIM), (8, OUT_DIM))
    b1 = jnp.broadcast_to(bias1.reshape(1, 256), (8, 256))
    b2 = jnp.broadcast_to(bias2.reshape(1, OUT_DIM), (8, OUT_DIM))

    tlo, thi, td1 = _prep1(x, W1, as1, ad1)
    t1 = jnp.concatenate([tlo, thi], axis=0)
    acc1 = _sc1(t1, td1, src, dst)
    t2, td2 = _mid(acc1[:N], acc1[N:], tlo, thi, td1, W2, e1, as2, ad2, b1)
    acc2 = _sc2(t2, td2, src, dst)
    return _final(acc2[:N], acc2[N:], t2, td2, b2)


# speedup vs baseline: 100.1680x; 1.7751x over previous
import functools

import jax
import jax.numpy as jnp
from jax import lax
from jax.experimental import pallas as pl
from jax.experimental.pallas import tpu as pltpu
from jax.experimental.pallas import tpu_sc as plsc

N = 10000
E = 320000
IN_DIM = 128
HID = 32
HEADS = 8
OUT_DIM = 64

ROW1 = 144
ROW2 = 80
CH = 80
NEG = 0.2
RPT = N // 16

_mesh = plsc.VectorSubcoreMesh(core_axis_name="c", subcore_axis_name="s")

def _take16(vec, idx):
    return jnp.take_along_axis(vec, idx, axis=0)


_B = 2000


def _prep1_body(x_ref, w_ref, as_ref, ad_ref, tlo_ref, thi_ref, td_ref):
    h = jnp.dot(x_ref[...], w_ref[...], preferred_element_type=jnp.float32)
    asrc = jnp.dot(h, as_ref[...], preferred_element_type=jnp.float32)
    adst = jnp.dot(h, ad_ref[...], preferred_element_type=jnp.float32)
    z12 = jnp.zeros((_B, 12), jnp.float32)
    tlo_ref[...] = jnp.concatenate([h[:, :128], asrc[:, 0:4], z12], axis=1)
    thi_ref[...] = jnp.concatenate([h[:, 128:], asrc[:, 4:8], z12], axis=1)
    td_ref[...] = jnp.concatenate([adst, jnp.zeros((_B, 8), jnp.float32)], axis=1)


def _prep1(x, w1, as1, ad1):
    return pl.pallas_call(
        _prep1_body,
        grid=(N // _B,),
        in_specs=[
            pl.BlockSpec((_B, IN_DIM), lambda i: (i, 0)),
            pl.BlockSpec((IN_DIM, 256), lambda i: (0, 0)),
            pl.BlockSpec((256, 8), lambda i: (0, 0)),
            pl.BlockSpec((256, 8), lambda i: (0, 0)),
        ],
        out_specs=[
            pl.BlockSpec((_B, ROW1), lambda i: (i, 0)),
            pl.BlockSpec((_B, ROW1), lambda i: (i, 0)),
            pl.BlockSpec((_B, 16), lambda i: (i, 0)),
        ],
        out_shape=[
            jax.ShapeDtypeStruct((N, ROW1), jnp.float32),
            jax.ShapeDtypeStruct((N, ROW1), jnp.float32),
            jax.ShapeDtypeStruct((N, 16), jnp.float32),
        ],
    )(x, w1, as1, ad1)


def _mid_body(alo_ref, ahi_ref, tlo_ref, thi_ref, td_ref, w2_ref, e1_ref,
              as2_ref, ad2_ref, b1_ref, t2_ref, td2_ref):
    asrc8 = jnp.concatenate([tlo_ref[:, 128:132], thi_ref[:, 128:132]], axis=1)
    adst8 = td_ref[:, 0:8]
    a = asrc8 + adst8
    wself = jnp.exp(jnp.where(a > 0, a, NEG * a))
    h1 = jnp.concatenate([tlo_ref[:, 0:128], thi_ref[:, 0:128]], axis=1)
    num = jnp.concatenate([alo_ref[:, 0:128], ahi_ref[:, 0:128]], axis=1)
    den8 = jnp.concatenate([alo_ref[:, 128:132], ahi_ref[:, 128:132]], axis=1)
    den8 = den8 + wself + 1e-16
    e1 = e1_ref[...]
    num = num + jnp.dot(wself, e1, preferred_element_type=jnp.float32) * h1
    den = jnp.dot(den8, e1, preferred_element_type=jnp.float32)
    out1 = jnp.maximum(num / den + b1_ref[0:1, :], 0.0)
    h2 = jnp.dot(out1, w2_ref[...], preferred_element_type=jnp.float32)
    asrc2 = jnp.sum(h2 * as2_ref[0:1, :], axis=1, keepdims=True)
    adst2 = jnp.sum(h2 * ad2_ref[0:1, :], axis=1, keepdims=True)
    z15 = jnp.zeros((_B, 15), jnp.float32)
    t2_ref[...] = jnp.concatenate([h2, asrc2, z15], axis=1)
    td2_ref[...] = jnp.concatenate([adst2, z15], axis=1)


def _mid(alo, ahi, tlo, thi, td1, w2, e1, as2, ad2, b1):
    return pl.pallas_call(
        _mid_body,
        grid=(N // _B,),
        in_specs=[
            pl.BlockSpec((_B, ROW1), lambda i: (i, 0)),
            pl.BlockSpec((_B, ROW1), lambda i: (i, 0)),
            pl.BlockSpec((_B, ROW1), lambda i: (i, 0)),
            pl.BlockSpec((_B, ROW1), lambda i: (i, 0)),
            pl.BlockSpec((_B, 16), lambda i: (i, 0)),
            pl.BlockSpec((256, OUT_DIM), lambda i: (0, 0)),
            pl.BlockSpec((8, 256), lambda i: (0, 0)),
            pl.BlockSpec((8, OUT_DIM), lambda i: (0, 0)),
            pl.BlockSpec((8, OUT_DIM), lambda i: (0, 0)),
            pl.BlockSpec((8, 256), lambda i: (0, 0)),
        ],
        out_specs=[
            pl.BlockSpec((_B, ROW2), lambda i: (i, 0)),
            pl.BlockSpec((_B, 16), lambda i: (i, 0)),
        ],
        out_shape=[
            jax.ShapeDtypeStruct((N, ROW2), jnp.float32),
            jax.ShapeDtypeStruct((N, 16), jnp.float32),
        ],
    )(alo, ahi, tlo, thi, td1, w2, e1, as2, ad2, b1)


def _final_body(a2lo_ref, a2hi_ref, t2_ref, td2_ref, b2_ref, out_ref):
    num = a2lo_ref[:, 0:64] + a2hi_ref[:, 0:64]
    den = a2lo_ref[:, 64:65] + a2hi_ref[:, 64:65]
    h2 = t2_ref[:, 0:64]
    a = t2_ref[:, 64:65] + td2_ref[:, 0:1]
    w = jnp.exp(jnp.where(a > 0, a, NEG * a))
    out_ref[...] = (num + w * h2) / (den + w + 1e-16) + b2_ref[0:1, :]


def _final(a2lo, a2hi, t2, td2, b2):
    return pl.pallas_call(
        _final_body,
        grid=(N // _B,),
        in_specs=[
            pl.BlockSpec((_B, ROW2), lambda i: (i, 0)),
            pl.BlockSpec((_B, ROW2), lambda i: (i, 0)),
            pl.BlockSpec((_B, ROW2), lambda i: (i, 0)),
            pl.BlockSpec((_B, 16), lambda i: (i, 0)),
            pl.BlockSpec((8, OUT_DIM), lambda i: (0, 0)),
        ],
        out_specs=pl.BlockSpec((_B, OUT_DIM), lambda i: (i, 0)),
        out_shape=jax.ShapeDtypeStruct((N, OUT_DIM), jnp.float32),
    )(a2lo, a2hi, t2, td2, b2)


def _zero_rows(buf, rows, width):
    z = jnp.zeros((16,), jnp.float32)

    def body(r, _):
        for v in range(width // 16):
            buf[r, pl.ds(16 * v, 16)] = z
        return 0

    lax.fori_loop(0, rows, body, 0)


def _zero_acc(gbuf, acc, base_r):
    for k in range(RPT // CH):
        pltpu.sync_copy(gbuf.at[pl.ds(0, CH)], acc.at[pl.ds(base_r + CH * k, CH)])
    rem = RPT - (RPT // CH) * CH
    if rem:
        pltpu.sync_copy(gbuf.at[pl.ds(0, rem)],
                        acc.at[pl.ds(base_r + RPT - rem, rem)])


def _make_sc(row, split_features, edge_fn):
    ept = E // 16 if split_features else E // 32
    nch = ept // CH

    @functools.partial(
        pl.kernel,
        out_type=jax.ShapeDtypeStruct((2 * N, row), jnp.float32),
        mesh=_mesh,
        compiler_params=pltpu.CompilerParams(use_tc_tiling_on_sc=False),
        scratch_types=[
            pltpu.VMEM((CH,), jnp.int32),
            pltpu.VMEM((CH,), jnp.int32),
            pltpu.VMEM((CH,), jnp.int32),
            pltpu.VMEM((CH,), jnp.int32),
            pltpu.VMEM((CH,), jnp.int32),
            pltpu.VMEM((CH,), jnp.int32),
            pltpu.VMEM((CH, row), jnp.float32),
            pltpu.VMEM((CH, row), jnp.float32),
            pltpu.VMEM((CH, 16), jnp.float32),
            pltpu.VMEM((CH, 16), jnp.float32),
            pltpu.VMEM_SHARED((N, row), jnp.float32),
            pltpu.SemaphoreType.DMA,
            pltpu.SemaphoreType.DMA,
            pltpu.SemaphoreType.DMA,
            pltpu.SemaphoreType.DMA,
            pltpu.SemaphoreType.DMA,
            pltpu.SemaphoreType.DMA,
            pltpu.SemaphoreType.DMA,
            pltpu.SemaphoreType.DMA,
        ],
    )
    def sc_pass(tbl, td, srcv, dstv, out, si0, si1, di0, di1, dc0, dc1,
                gbuf0, gbuf1, dbuf0, dbuf1, acc,
                gs0, gs1, ds0, ds1, ss0, ss1, is0, is1):
        c = lax.axis_index("c")
        s = lax.axis_index("s")
        sib, dib, dsc = (si0, si1), (di0, di1), (dc0, dc1)
        gbufs, dbufs = (gbuf0, gbuf1), (dbuf0, dbuf1)
        gsems, dsems, ssems, isems = (gs0, gs1), (ds0, ds1), (ss0, ss1), (is0, is1)

        tile_e0 = (s * ept) if split_features else ((s * 2 + c) * ept)
        tile_e0 = pl.multiple_of(tile_e0, 8)

        _zero_rows(gbuf0, CH, row)
        _zero_acc(gbuf0, acc, s * RPT)
        plsc.subcore_barrier()

        off = c * N if split_features else 0

        def start_idx(j, b):
            e0 = pl.multiple_of(tile_e0 + j * CH, 8)
            pltpu.async_copy(srcv.at[pl.ds(e0, CH)], sib[b], isems[b])
            pltpu.async_copy(dstv.at[pl.ds(e0, CH)], dib[b], isems[b])

        def wait_idx_adjust(j, b):
            e0 = pl.multiple_of(tile_e0 + j * CH, 8)
            pltpu.make_async_copy(srcv.at[pl.ds(e0, CH)], sib[b],
                                  isems[b]).wait()
            pltpu.make_async_copy(dstv.at[pl.ds(e0, CH)], dib[b],
                                  isems[b]).wait()
            if split_features:
                for k in range(CH // 16):
                    sib[b][pl.ds(k * 16, 16)] = sib[b][pl.ds(k * 16, 16)] + off

        def start_gather(j, b):
            pltpu.async_copy(tbl.at[sib[b]], gbufs[b], gsems[b])
            pltpu.async_copy(td.at[dib[b]], dbufs[b], dsems[b])

        def wait_gather(j, b):
            pltpu.make_async_copy(tbl.at[sib[b]], gbufs[b], gsems[b]).wait()
            pltpu.make_async_copy(td.at[dib[b]], dbufs[b], dsems[b]).wait()

        def start_scatter(j, b):
            pltpu.async_copy(gbufs[b], acc.at[dsc[b]], ssems[b], add=True)

        def wait_scatter(j, b):
            pltpu.make_async_copy(gbufs[b], acc.at[dsc[b]], ssems[b]).wait()

        def keep_scatter_idx(b):
            for k in range(CH // 16):
                dsc[b][pl.ds(k * 16, 16)] = dib[b][pl.ds(k * 16, 16)]

        lane = lax.iota(jnp.int32, 16)
        consts = (lane, c)

        pltpu.sync_copy(srcv.at[pl.ds(tile_e0, CH)], si0)
        pltpu.sync_copy(dstv.at[pl.ds(tile_e0, CH)], di0)
        if split_features:
            for k in range(CH // 16):
                si0[pl.ds(k * 16, 16)] = si0[pl.ds(k * 16, 16)] + off
        start_gather(0, 0)
        start_idx(1, 1)

        def pair(jj, _):
            for b in (0, 1):
                j = 2 * jj + b

                @pl.when(j >= 1)
                def _():
                    wait_scatter(j - 1, 1 - b)

                @pl.when(j + 1 < nch)
                def _():
                    wait_idx_adjust(j + 1, 1 - b)
                    start_gather(j + 1, 1 - b)

                wait_gather(j, b)
                keep_scatter_idx(b)

                @pl.when(j + 2 < nch)
                def _():
                    start_idx(j + 2, b)

                edge_fn(gbufs[b], dbufs[b], consts)
                start_scatter(j, b)
            return 0

        lax.fori_loop(0, nch // 2, pair, 0)
        if nch % 2:
            jl = nch - 1
            wait_scatter(jl - 1, 1)
            wait_gather(jl, 0)
            keep_scatter_idx(0)
            edge_fn(gbufs[0], dbufs[0], consts)
            start_scatter(jl, 0)
            wait_scatter(jl, 0)
        else:
            wait_scatter(nch - 1, 1)
        plsc.subcore_barrier()
        pltpu.sync_copy(acc.at[pl.ds(s * RPT, RPT)],
                        out.at[pl.ds(c * N + s * RPT, RPT)])

    return sc_pass


def _edges1(gbuf, dbuf, consts):
    lane, c = consts
    shift_idx = (lane + 4 * c) & 15
    head_idx = [jnp.full((16,), hh, jnp.int32) for hh in range(4)]

    @functools.partial(plsc.parallel_loop, 0, CH, unroll=4)
    def edge(e):
        adsh = _take16(dbuf[e, pl.ds(0, 16)], shift_idx)
        a = gbuf[e, pl.ds(128, 16)] + adsh
        a = jnp.where(a > 0, a, NEG * a)
        w = jnp.exp(a)
        gbuf[e, pl.ds(128, 16)] = jnp.where(lane < 4, w, 0.0)
        for hh in range(4):
            ws = _take16(w, head_idx[hh])
            for v in (2 * hh, 2 * hh + 1):
                gbuf[e, pl.ds(16 * v, 16)] = gbuf[e, pl.ds(16 * v, 16)] * ws


def _edges2(gbuf, dbuf, consts):
    lane, _ = consts
    zidx = jnp.zeros((16,), jnp.int32)

    @functools.partial(plsc.parallel_loop, 0, CH, unroll=4)
    def edge(e):
        a = gbuf[e, pl.ds(64, 16)] + dbuf[e, pl.ds(0, 16)]
        a = jnp.where(a > 0, a, NEG * a)
        w = jnp.exp(a)
        gbuf[e, pl.ds(64, 16)] = jnp.where(lane < 1, w, 0.0)
        ws = _take16(w, zidx)
        for v in range(4):
            gbuf[e, pl.ds(16 * v, 16)] = gbuf[e, pl.ds(16 * v, 16)] * ws


_sc1 = _make_sc(ROW1, True, _edges1)
_sc2 = _make_sc(ROW2, False, _edges2)


def kernel(x, edge_index, W1, att_src1, att_dst1, bias1,
           W2, att_src2, att_dst2, bias2):
    src = edge_index[0]
    dst = edge_index[1]
    dst2d = dst.reshape(E // CH, CH)

    e1 = (jnp.arange(256, dtype=jnp.int32) // HID ==
          jnp.arange(HEADS, dtype=jnp.int32)[:, None]).astype(jnp.float32)
    as1 = e1.T * att_src1.reshape(256)[:, None]
    ad1 = e1.T * att_dst1.reshape(256)[:, None]
    as2 = jnp.broadcast_to(att_src2.reshape(1, OUT_DIM), (8, OUT_DIM))
    ad2 = jnp.broadcast_to(att_dst2.reshape(1, OUT_DIM), (8, OUT_DIM))
    b1 = jnp.broadcast_to(bias1.reshape(1, 256), (8, 256))
    b2 = jnp.broadcast_to(bias2.reshape(1, OUT_DIM), (8, OUT_DIM))

    tlo, thi, td1 = _prep1(x, W1, as1, ad1)
    t1 = jnp.concatenate([tlo, thi], axis=0)
    acc1 = _sc1(t1, td1, src, dst)
    t2, td2 = _mid(acc1[:N], acc1[N:], tlo, thi, td1, W2, e1, as2, ad2, b1)
    acc2 = _sc2(t2, td2, src, dst)
    return _final(acc2[:N], acc2[N:], t2, td2, b2)
